# Initial kernel scaffold; baseline (speedup 1.0000x reference)
import jax
import jax.numpy as jnp
from jax.experimental import pallas as pl


def kernel(query, memory_keys, Wq, bq, Wm, bm, ln_scale, ln_bias, Wc, bc, k):
    raise NotImplementedError("write your pallas kernel here")


# speedup vs baseline: 1.7833x; 1.7833x over previous
import functools

import jax
import jax.numpy as jnp
from jax import lax
from jax.experimental import pallas as pl
from jax.experimental.pallas import tpu as pltpu
from jax.experimental.pallas import tpu_sc as plsc

_TOPK = 50
_CW = 128
_QT = 256
_KB = 2048
_QT2 = 128
_QT3 = 128
_NEG = -3.0e38


def _stage1_body(K, q_ref, m_ref, vals_ref, idx_ref):
    ik = pl.program_id(1)
    q = jnp.maximum(q_ref[...], 0.0)
    qn = q / jnp.maximum(jnp.sqrt(jnp.sum(q * q, axis=1, keepdims=True)), 1e-8)
    m = m_ref[...]
    mn = m / jnp.maximum(jnp.sqrt(jnp.sum(m * m, axis=1, keepdims=True)), 1e-8)
    sim = lax.dot_general(qn, mn, (((1,), (1,)), ((), ())),
                          preferred_element_type=jnp.float32)
    iot = lax.broadcasted_iota(jnp.int32, sim.shape, 1)
    col = ik * _KB + iot
    sim = jnp.where(col < K, sim, _NEG)
    base = ik * _KB
    vals_ref[...] = jnp.full((sim.shape[0], _CW), _NEG, jnp.float32)
    idx_ref[...] = jnp.zeros((sim.shape[0], _CW), jnp.int32)
    for i in range(_TOPK):
        mval = jnp.max(sim, axis=1, keepdims=True)
        cand = jnp.where(sim == mval, iot, _KB)
        am = jnp.min(cand, axis=1, keepdims=True)
        vals_ref[:, i:i + 1] = mval
        idx_ref[:, i:i + 1] = base + am
        sim = jnp.where(iot == am, _NEG, sim)


def _merge_body(ncand, vals_ref, idx_ref, oidx_ref):
    v = vals_ref[...]
    ids = idx_ref[...]
    iot = lax.broadcasted_iota(jnp.int32, v.shape, 1)
    for i in range(_TOPK):
        mval = jnp.max(v, axis=1, keepdims=True)
        cand = jnp.where(v == mval, iot, ncand)
        am = jnp.min(cand, axis=1, keepdims=True)
        gi = jnp.max(jnp.where(iot == am, ids, -1), axis=1, keepdims=True)
        oidx_ref[:, i:i + 1] = gi
        v = jnp.where(iot == am, _NEG, v)


def _sc_gather(table, idx_flat):
    info = plsc.get_sparse_core_info()
    nw = info.num_cores * info.num_subcores
    b_total = idx_flat.shape[0]
    d = table.shape[1]
    bpw = b_total // nw
    chunk = 80
    nchunk = bpw // chunk
    mesh = plsc.VectorSubcoreMesh(core_axis_name="c", subcore_axis_name="s")

    @functools.partial(
        pl.kernel, mesh=mesh,
        out_type=jax.ShapeDtypeStruct((b_total, d), jnp.float32),
        scratch_types=[
            pltpu.VMEM((bpw,), jnp.int32),
            pltpu.VMEM((bpw, d), jnp.float32),
            pltpu.SemaphoreType.DMA,
        ],
        compiler_params=pltpu.CompilerParams(use_tc_tiling_on_sc=False),
    )
    def gather_kernel(table_hbm, idx_hbm, out_hbm, idx_v, rows_v, sem):
        wid = lax.axis_index("s") * info.num_cores + lax.axis_index("c")
        base = wid * bpw
        pltpu.sync_copy(idx_hbm.at[pl.ds(base, bpw)], idx_v)
        copies = []
        for c in range(nchunk):
            copies.append(pltpu.async_copy(
                table_hbm.at[idx_v.at[pl.ds(c * chunk, chunk)]],
                rows_v.at[pl.ds(c * chunk, chunk)], sem))
        for cp in copies:
            cp.wait()
        pltpu.sync_copy(rows_v, out_hbm.at[pl.ds(base, bpw)])

    return gather_kernel(table, idx_flat)


def _post_body(g_ref, q_ref, wqt_ref, bq_ref, wmt_ref, bm_ref, lns_ref,
               lnb_ref, wcq_ref, wco_ref, bc_ref, out_ref):
    q = jnp.maximum(q_ref[...], 0.0)
    qt = jnp.maximum(
        jnp.dot(q, wqt_ref[...], preferred_element_type=jnp.float32,
                precision=lax.Precision.HIGHEST) + bq_ref[...], 0.0)
    x = g_ref[...].reshape(_TOPK * _QT3, q.shape[1])
    mt = jnp.maximum(
        jnp.dot(x, wmt_ref[...], preferred_element_type=jnp.float32,
                precision=lax.Precision.HIGHEST) + bm_ref[...], 0.0)
    acc = qt
    for j in range(_TOPK):
        acc = acc + mt[j * _QT3:(j + 1) * _QT3, :]
    mu = jnp.mean(acc, axis=1, keepdims=True)
    var = jnp.mean((acc - mu) ** 2, axis=1, keepdims=True)
    out = (acc - mu) / jnp.sqrt(var + 1e-5) * lns_ref[...] + lnb_ref[...]
    logits = (jnp.dot(q, wcq_ref[...], preferred_element_type=jnp.float32,
                      precision=lax.Precision.HIGHEST)
              + jnp.dot(out, wco_ref[...], preferred_element_type=jnp.float32,
                        precision=lax.Precision.HIGHEST)
              + bc_ref[...])
    out_ref[...] = logits


def kernel(query, memory_keys, Wq, bq, Wm, bm, ln_scale, ln_bias, Wc, bc, k):
    Q, D = query.shape
    K = memory_keys.shape[0]
    U = Wq.shape[0]
    C = Wc.shape[0]
    nkb = -(-K // _KB)
    ncand = nkb * _CW

    vals, idxs = pl.pallas_call(
        functools.partial(_stage1_body, K),
        grid=(Q // _QT, nkb),
        in_specs=[
            pl.BlockSpec((_QT, D), lambda i, j: (i, 0)),
            pl.BlockSpec((_KB, D), lambda i, j: (j, 0)),
        ],
        out_specs=[
            pl.BlockSpec((_QT, _CW), lambda i, j: (i, j)),
            pl.BlockSpec((_QT, _CW), lambda i, j: (i, j)),
        ],
        out_shape=[
            jax.ShapeDtypeStruct((Q, ncand), jnp.float32),
            jax.ShapeDtypeStruct((Q, ncand), jnp.int32),
        ],
        compiler_params=pltpu.CompilerParams(
            dimension_semantics=("parallel", "parallel")),
    )(query, memory_keys)

    topi = pl.pallas_call(
        functools.partial(_merge_body, ncand),
        grid=(Q // _QT2,),
        in_specs=[
            pl.BlockSpec((_QT2, ncand), lambda i: (i, 0)),
            pl.BlockSpec((_QT2, ncand), lambda i: (i, 0)),
        ],
        out_specs=pl.BlockSpec((_QT2, _TOPK), lambda i: (i, 0)),
        out_shape=jax.ShapeDtypeStruct((Q, _TOPK), jnp.int32),
    )(vals, idxs)

    idx_flat = jnp.transpose(topi).reshape(-1)
    g = _sc_gather(memory_keys, idx_flat)
    g3 = g.reshape(_TOPK, Q, D)

    logits = pl.pallas_call(
        _post_body,
        grid=(Q // _QT3,),
        in_specs=[
            pl.BlockSpec((_TOPK, _QT3, D), lambda i: (0, i, 0)),
            pl.BlockSpec((_QT3, D), lambda i: (i, 0)),
            pl.BlockSpec((D, U), lambda i: (0, 0)),
            pl.BlockSpec((1, U), lambda i: (0, 0)),
            pl.BlockSpec((D, U), lambda i: (0, 0)),
            pl.BlockSpec((1, U), lambda i: (0, 0)),
            pl.BlockSpec((1, U), lambda i: (0, 0)),
            pl.BlockSpec((1, U), lambda i: (0, 0)),
            pl.BlockSpec((D, C), lambda i: (0, 0)),
            pl.BlockSpec((U, C), lambda i: (0, 0)),
            pl.BlockSpec((1, C), lambda i: (0, 0)),
        ],
        out_specs=pl.BlockSpec((_QT3, C), lambda i: (i, 0)),
        out_shape=jax.ShapeDtypeStruct((Q, C), jnp.float32),
    )(g3, query, jnp.transpose(Wq), bq[None, :], jnp.transpose(Wm),
      bm[None, :], ln_scale[None, :], ln_bias[None, :],
      jnp.transpose(Wc[:, :D]), jnp.transpose(Wc[:, D:]), bc[None, :])
    return logits


# speedup vs baseline: 4.0328x; 2.2614x over previous
import functools

import jax
import jax.numpy as jnp
from jax import lax
from jax.experimental import pallas as pl
from jax.experimental.pallas import tpu as pltpu
from jax.experimental.pallas import tpu_sc as plsc

_TOPK = 50
_MFAST = 16
_CW = 128
_QT = 256
_KB = 2048
_QT2 = 128
_QT3 = 128
_NEG = -3.0e38


def _stage1_body(K, M, q_ref, m_ref, vals_ref, idx_ref):
    ik = pl.program_id(1)
    q = jnp.maximum(q_ref[...], 0.0)
    qn = q / jnp.maximum(jnp.sqrt(jnp.sum(q * q, axis=1, keepdims=True)), 1e-8)
    m = m_ref[...]
    mn = m / jnp.maximum(jnp.sqrt(jnp.sum(m * m, axis=1, keepdims=True)), 1e-8)
    sim = lax.dot_general(qn, mn, (((1,), (1,)), ((), ())),
                          preferred_element_type=jnp.float32)
    iot = lax.broadcasted_iota(jnp.int32, sim.shape, 1)
    col = ik * _KB + iot
    sim = jnp.where(col < K, sim, _NEG)
    base = ik * _KB
    idx_ref[...] = jnp.zeros((sim.shape[0], _CW), jnp.int32)
    mval = None
    for i in range(M):
        mval = jnp.max(sim, axis=1, keepdims=True)
        cand = jnp.where(sim == mval, iot, _KB)
        am = jnp.min(cand, axis=1, keepdims=True)
        vals_ref[:, i:i + 1] = mval
        idx_ref[:, i:i + 1] = base + am
        sim = jnp.where(cand == am, _NEG, sim)
    vals_ref[:, M:] = jnp.broadcast_to(mval, (sim.shape[0], _CW - M))


def _merge_body(ncand, M, vals_ref, idx_ref, oidx_ref, flag_ref=None):
    v_raw = vals_ref[...]
    ids = idx_ref[...]
    iot = lax.broadcasted_iota(jnp.int32, v_raw.shape, 1)
    lanemod = iot & (_CW - 1)
    if flag_ref is not None:
        tmax = jnp.max(jnp.where(lanemod >= M, v_raw, _NEG), axis=1,
                       keepdims=True)
    v = jnp.where(lanemod < M, v_raw, _NEG)
    mval = None
    for i in range(_TOPK):
        mval = jnp.max(v, axis=1, keepdims=True)
        cand = jnp.where(v == mval, iot, ncand)
        am = jnp.min(cand, axis=1, keepdims=True)
        gi = jnp.max(jnp.where(iot == am, ids, -1), axis=1, keepdims=True)
        oidx_ref[:, i:i + 1] = gi
        v = jnp.where(iot == am, _NEG, v)
    if flag_ref is not None:
        flag_ref[...] = jnp.broadcast_to(
            (tmax >= mval).astype(jnp.int32), flag_ref.shape)


def _sc_gather(table, idx_flat):
    info = plsc.get_sparse_core_info()
    nw = info.num_cores * info.num_subcores
    b_total = idx_flat.shape[0]
    d = table.shape[1]
    bpw = b_total // nw
    chunk = 80
    nchunk = bpw // chunk
    mesh = plsc.VectorSubcoreMesh(core_axis_name="c", subcore_axis_name="s")

    @functools.partial(
        pl.kernel, mesh=mesh,
        out_type=jax.ShapeDtypeStruct((b_total, d), jnp.float32),
        scratch_types=[
            pltpu.VMEM((bpw,), jnp.int32),
            pltpu.VMEM((bpw, d), jnp.float32),
            pltpu.SemaphoreType.DMA,
        ],
        compiler_params=pltpu.CompilerParams(use_tc_tiling_on_sc=False),
    )
    def gather_kernel(table_hbm, idx_hbm, out_hbm, idx_v, rows_v, sem):
        wid = lax.axis_index("s") * info.num_cores + lax.axis_index("c")
        base = wid * bpw
        pltpu.sync_copy(idx_hbm.at[pl.ds(base, bpw)], idx_v)
        copies = []
        for c in range(nchunk):
            copies.append(pltpu.async_copy(
                table_hbm.at[idx_v.at[pl.ds(c * chunk, chunk)]],
                rows_v.at[pl.ds(c * chunk, chunk)], sem))
        for cp in copies:
            cp.wait()
        pltpu.sync_copy(rows_v, out_hbm.at[pl.ds(base, bpw)])

    return gather_kernel(table, idx_flat)


def _post_body(g_ref, q_ref, wqt_ref, bq_ref, wmt_ref, bm_ref, lns_ref,
               lnb_ref, wcq_ref, wco_ref, bc_ref, out_ref):
    q = jnp.maximum(q_ref[...], 0.0)
    qt = jnp.maximum(
        jnp.dot(q, wqt_ref[...], preferred_element_type=jnp.float32,
                precision=lax.Precision.HIGHEST) + bq_ref[...], 0.0)
    x = g_ref[...].reshape(_TOPK * _QT3, q.shape[1])
    mt = jnp.maximum(
        jnp.dot(x, wmt_ref[...], preferred_element_type=jnp.float32,
                precision=lax.Precision.HIGHEST) + bm_ref[...], 0.0)
    acc = qt
    for j in range(_TOPK):
        acc = acc + mt[j * _QT3:(j + 1) * _QT3, :]
    mu = jnp.mean(acc, axis=1, keepdims=True)
    var = jnp.mean((acc - mu) ** 2, axis=1, keepdims=True)
    out = (acc - mu) / jnp.sqrt(var + 1e-5) * lns_ref[...] + lnb_ref[...]
    logits = (jnp.dot(q, wcq_ref[...], preferred_element_type=jnp.float32,
                      precision=lax.Precision.HIGHEST)
              + jnp.dot(out, wco_ref[...], preferred_element_type=jnp.float32,
                        precision=lax.Precision.HIGHEST)
              + bc_ref[...])
    out_ref[...] = logits


def kernel(query, memory_keys, Wq, bq, Wm, bm, ln_scale, ln_bias, Wc, bc, k):
    Q, D = query.shape
    K = memory_keys.shape[0]
    U = Wq.shape[0]
    C = Wc.shape[0]
    nkb = -(-K // _KB)
    ncand = nkb * _CW

    def run_stage1(M):
        return pl.pallas_call(
            functools.partial(_stage1_body, K, M),
            grid=(Q // _QT, nkb),
            in_specs=[
                pl.BlockSpec((_QT, D), lambda i, j: (i, 0)),
                pl.BlockSpec((_KB, D), lambda i, j: (j, 0)),
            ],
            out_specs=[
                pl.BlockSpec((_QT, _CW), lambda i, j: (i, j)),
                pl.BlockSpec((_QT, _CW), lambda i, j: (i, j)),
            ],
            out_shape=[
                jax.ShapeDtypeStruct((Q, ncand), jnp.float32),
                jax.ShapeDtypeStruct((Q, ncand), jnp.int32),
            ],
            compiler_params=pltpu.CompilerParams(
                dimension_semantics=("parallel", "parallel")),
        )(query, memory_keys)

    def run_merge(vals, idxs, M, certify):
        out_shape = [jax.ShapeDtypeStruct((Q, _TOPK), jnp.int32)]
        out_specs = [pl.BlockSpec((_QT2, _TOPK), lambda i: (i, 0))]
        if certify:
            out_shape.append(jax.ShapeDtypeStruct((Q, 128), jnp.int32))
            out_specs.append(pl.BlockSpec((_QT2, 128), lambda i: (i, 0)))
        return pl.pallas_call(
            functools.partial(_merge_body, ncand, M),
            grid=(Q // _QT2,),
            in_specs=[
                pl.BlockSpec((_QT2, ncand), lambda i: (i, 0)),
                pl.BlockSpec((_QT2, ncand), lambda i: (i, 0)),
            ],
            out_specs=out_specs,
            out_shape=out_shape,
        )(vals, idxs)

    vals_f, idxs_f = run_stage1(_MFAST)
    topi_fast, flags = run_merge(vals_f, idxs_f, _MFAST, certify=True)

    def full_path():
        vals50, idxs50 = run_stage1(_TOPK)
        (topi50,) = run_merge(vals50, idxs50, _TOPK, certify=False)
        return topi50

    topi = lax.cond(jnp.max(flags) > 0, full_path, lambda: topi_fast)

    idx_flat = jnp.transpose(topi).reshape(-1)
    g = _sc_gather(memory_keys, idx_flat)
    g3 = g.reshape(_TOPK, Q, D)

    logits = pl.pallas_call(
        _post_body,
        grid=(Q // _QT3,),
        in_specs=[
            pl.BlockSpec((_TOPK, _QT3, D), lambda i: (0, i, 0)),
            pl.BlockSpec((_QT3, D), lambda i: (i, 0)),
            pl.BlockSpec((D, U), lambda i: (0, 0)),
            pl.BlockSpec((1, U), lambda i: (0, 0)),
            pl.BlockSpec((D, U), lambda i: (0, 0)),
            pl.BlockSpec((1, U), lambda i: (0, 0)),
            pl.BlockSpec((1, U), lambda i: (0, 0)),
            pl.BlockSpec((1, U), lambda i: (0, 0)),
            pl.BlockSpec((D, C), lambda i: (0, 0)),
            pl.BlockSpec((U, C), lambda i: (0, 0)),
            pl.BlockSpec((1, C), lambda i: (0, 0)),
        ],
        out_specs=pl.BlockSpec((_QT3, C), lambda i: (i, 0)),
        out_shape=jax.ShapeDtypeStruct((Q, C), jnp.float32),
    )(g3, query, jnp.transpose(Wq), bq[None, :], jnp.transpose(Wm),
      bm[None, :], ln_scale[None, :], ln_bias[None, :],
      jnp.transpose(Wc[:, :D]), jnp.transpose(Wc[:, D:]), bc[None, :])
    return logits


# speedup vs baseline: 8.0932x; 2.0068x over previous
import functools

import jax
import jax.numpy as jnp
from jax import lax
from jax.experimental import pallas as pl
from jax.experimental.pallas import tpu as pltpu
from jax.experimental.pallas import tpu_sc as plsc

_TOPK = 50
_MFAST = 8
_QT = 256
_KB = 2048
_QT2 = 128
_QT3 = 128
_NEG = -3.0e38


def _stage1_body(K, M, q_ref, m_ref, vals_ref, idx_ref):
    ik = pl.program_id(1)
    q = jnp.maximum(q_ref[...], 0.0)
    qn = q / jnp.maximum(jnp.sqrt(jnp.sum(q * q, axis=1, keepdims=True)), 1e-8)
    m = m_ref[...]
    mn = m / jnp.maximum(jnp.sqrt(jnp.sum(m * m, axis=1, keepdims=True)), 1e-8)
    sim = lax.dot_general(qn, mn, (((1,), (1,)), ((), ())),
                          preferred_element_type=jnp.float32)
    iot = lax.broadcasted_iota(jnp.int32, sim.shape, 1)
    col = ik * _KB + iot
    sim = jnp.where(col < K, sim, _NEG)
    base = ik * _KB
    for i in range(M):
        mval = jnp.max(sim, axis=1, keepdims=True)
        cand = jnp.where(sim == mval, iot, _KB)
        am = jnp.min(cand, axis=1, keepdims=True)
        vals_ref[0, :, i:i + 1] = mval
        idx_ref[0, :, i:i + 1] = base + am
        sim = jnp.where(cand == am, _NEG, sim)


def _merge_body(ncand, M, vals_ref, idx_ref, oidx_ref, flag_ref=None):
    v = vals_ref[...]
    ids = idx_ref[...]
    iot = lax.broadcasted_iota(jnp.int32, v.shape, 1)
    if flag_ref is not None:
        tmax = jnp.max(jnp.where((iot & (M - 1)) == M - 1, v, _NEG), axis=1,
                       keepdims=True)
    mval = None
    for i in range(_TOPK):
        mval = jnp.max(v, axis=1, keepdims=True)
        cand = jnp.where(v == mval, iot, ncand)
        am = jnp.min(cand, axis=1, keepdims=True)
        gi = jnp.max(jnp.where(iot == am, ids, -1), axis=1, keepdims=True)
        oidx_ref[:, i:i + 1] = gi
        v = jnp.where(iot == am, _NEG, v)
    if flag_ref is not None:
        flag_ref[...] = jnp.broadcast_to(
            (tmax >= mval).astype(jnp.int32), flag_ref.shape)


def _sc_gather(table, idx_flat):
    info = plsc.get_sparse_core_info()
    nw = info.num_cores * info.num_subcores
    b_total = idx_flat.shape[0]
    d = table.shape[1]
    bpw = b_total // nw
    chunk = 80
    nchunk = bpw // chunk
    mesh = plsc.VectorSubcoreMesh(core_axis_name="c", subcore_axis_name="s")

    @functools.partial(
        pl.kernel, mesh=mesh,
        out_type=jax.ShapeDtypeStruct((b_total, d), jnp.float32),
        scratch_types=[
            pltpu.VMEM((bpw,), jnp.int32),
            pltpu.VMEM((bpw, d), jnp.float32),
            pltpu.SemaphoreType.DMA,
        ],
        compiler_params=pltpu.CompilerParams(use_tc_tiling_on_sc=False),
    )
    def gather_kernel(table_hbm, idx_hbm, out_hbm, idx_v, rows_v, sem):
        wid = lax.axis_index("s") * info.num_cores + lax.axis_index("c")
        base = wid * bpw
        pltpu.sync_copy(idx_hbm.at[pl.ds(base, bpw)], idx_v)
        copies = []
        for c in range(nchunk):
            copies.append(pltpu.async_copy(
                table_hbm.at[idx_v.at[pl.ds(c * chunk, chunk)]],
                rows_v.at[pl.ds(c * chunk, chunk)], sem))
        for cp in copies:
            cp.wait()
        pltpu.sync_copy(rows_v, out_hbm.at[pl.ds(base, bpw)])

    return gather_kernel(table, idx_flat)


def _post_body(g_ref, q_ref, wqt_ref, bq_ref, wmt_ref, bm_ref, lns_ref,
               lnb_ref, wcq_ref, wco_ref, bc_ref, out_ref):
    q = jnp.maximum(q_ref[...], 0.0)
    qt = jnp.maximum(
        jnp.dot(q, wqt_ref[...], preferred_element_type=jnp.float32,
                precision=lax.Precision.HIGHEST) + bq_ref[...], 0.0)
    x = g_ref[...].reshape(_TOPK * _QT3, q.shape[1])
    mt = jnp.maximum(
        jnp.dot(x, wmt_ref[...], preferred_element_type=jnp.float32,
                precision=lax.Precision.HIGHEST) + bm_ref[...], 0.0)
    acc = qt
    for j in range(_TOPK):
        acc = acc + mt[j * _QT3:(j + 1) * _QT3, :]
    mu = jnp.mean(acc, axis=1, keepdims=True)
    var = jnp.mean((acc - mu) ** 2, axis=1, keepdims=True)
    out = (acc - mu) / jnp.sqrt(var + 1e-5) * lns_ref[...] + lnb_ref[...]
    logits = (jnp.dot(q, wcq_ref[...], preferred_element_type=jnp.float32,
                      precision=lax.Precision.HIGHEST)
              + jnp.dot(out, wco_ref[...], preferred_element_type=jnp.float32,
                        precision=lax.Precision.HIGHEST)
              + bc_ref[...])
    out_ref[...] = logits


def kernel(query, memory_keys, Wq, bq, Wm, bm, ln_scale, ln_bias, Wc, bc, k):
    Q, D = query.shape
    K = memory_keys.shape[0]
    U = Wq.shape[0]
    C = Wc.shape[0]
    nkb = -(-K // _KB)

    def run_stage1(M):
        return pl.pallas_call(
            functools.partial(_stage1_body, K, M),
            grid=(Q // _QT, nkb),
            in_specs=[
                pl.BlockSpec((_QT, D), lambda i, j: (i, 0)),
                pl.BlockSpec((_KB, D), lambda i, j: (j, 0)),
            ],
            out_specs=[
                pl.BlockSpec((1, _QT, M), lambda i, j: (j, i, 0)),
                pl.BlockSpec((1, _QT, M), lambda i, j: (j, i, 0)),
            ],
            out_shape=[
                jax.ShapeDtypeStruct((nkb, Q, M), jnp.float32),
                jax.ShapeDtypeStruct((nkb, Q, M), jnp.int32),
            ],
            compiler_params=pltpu.CompilerParams(
                dimension_semantics=("parallel", "parallel")),
        )(query, memory_keys)

    def run_merge(vals, idxs, M, certify):
        ncand = nkb * M
        vals = jnp.swapaxes(vals, 0, 1).reshape(Q, ncand)
        idxs = jnp.swapaxes(idxs, 0, 1).reshape(Q, ncand)
        out_shape = [jax.ShapeDtypeStruct((Q, _TOPK), jnp.int32)]
        out_specs = [pl.BlockSpec((_QT2, _TOPK), lambda i: (i, 0))]
        if certify:
            out_shape.append(jax.ShapeDtypeStruct((Q, 128), jnp.int32))
            out_specs.append(pl.BlockSpec((_QT2, 128), lambda i: (i, 0)))
        return pl.pallas_call(
            functools.partial(_merge_body, ncand, M),
            grid=(Q // _QT2,),
            in_specs=[
                pl.BlockSpec((_QT2, ncand), lambda i: (i, 0)),
                pl.BlockSpec((_QT2, ncand), lambda i: (i, 0)),
            ],
            out_specs=out_specs,
            out_shape=out_shape,
        )(vals, idxs)

    vals_f, idxs_f = run_stage1(_MFAST)
    topi_fast, flags = run_merge(vals_f, idxs_f, _MFAST, certify=True)

    def full_path():
        vals50, idxs50 = run_stage1(_TOPK)
        (topi50,) = run_merge(vals50, idxs50, _TOPK, certify=False)
        return topi50

    topi = lax.cond(jnp.max(flags) > 0, full_path, lambda: topi_fast)

    idx_flat = jnp.transpose(topi).reshape(-1)
    g = _sc_gather(memory_keys, idx_flat)
    g3 = g.reshape(_TOPK, Q, D)

    logits = pl.pallas_call(
        _post_body,
        grid=(Q // _QT3,),
        in_specs=[
            pl.BlockSpec((_TOPK, _QT3, D), lambda i: (0, i, 0)),
            pl.BlockSpec((_QT3, D), lambda i: (i, 0)),
            pl.BlockSpec((D, U), lambda i: (0, 0)),
            pl.BlockSpec((1, U), lambda i: (0, 0)),
            pl.BlockSpec((D, U), lambda i: (0, 0)),
            pl.BlockSpec((1, U), lambda i: (0, 0)),
            pl.BlockSpec((1, U), lambda i: (0, 0)),
            pl.BlockSpec((1, U), lambda i: (0, 0)),
            pl.BlockSpec((D, C), lambda i: (0, 0)),
            pl.BlockSpec((U, C), lambda i: (0, 0)),
            pl.BlockSpec((1, C), lambda i: (0, 0)),
        ],
        out_specs=pl.BlockSpec((_QT3, C), lambda i: (i, 0)),
        out_shape=jax.ShapeDtypeStruct((Q, C), jnp.float32),
    )(g3, query, jnp.transpose(Wq), bq[None, :], jnp.transpose(Wm),
      bm[None, :], ln_scale[None, :], ln_bias[None, :],
      jnp.transpose(Wc[:, :D]), jnp.transpose(Wc[:, D:]), bc[None, :])
    return logits


# speedup vs baseline: 8.2880x; 1.0241x over previous
import functools

import jax
import jax.numpy as jnp
from jax import lax
from jax.experimental import pallas as pl
from jax.experimental.pallas import tpu as pltpu
from jax.experimental.pallas import tpu_sc as plsc

_TOPK = 50
_MFAST = 8
_QT = 512
_KB = 2048
_QT2 = 128
_QT3 = 128
_NEG = -3.0e38


def _stage1_body(K, M, nkb, q_ref, m_ref, vals_ref, idx_ref):
    ik = pl.program_id(1)
    q = jnp.maximum(q_ref[...], 0.0)
    qn = q / jnp.maximum(jnp.sqrt(jnp.sum(q * q, axis=1, keepdims=True)), 1e-8)
    m = m_ref[...]
    mn = m / jnp.maximum(jnp.sqrt(jnp.sum(m * m, axis=1, keepdims=True)), 1e-8)
    sim = lax.dot_general(qn, mn, (((1,), (1,)), ((), ())),
                          preferred_element_type=jnp.float32)
    iot = lax.broadcasted_iota(jnp.int32, sim.shape, 1)
    sim = lax.cond(
        ik == nkb - 1,
        lambda: jnp.where(ik * _KB + iot < K, sim, _NEG),
        lambda: sim)
    base = ik * _KB
    for i in range(M):
        mval = jnp.max(sim, axis=1, keepdims=True)
        cand = jnp.where(sim == mval, iot, _KB)
        am = jnp.min(cand, axis=1, keepdims=True)
        vals_ref[0, :, i:i + 1] = mval
        idx_ref[0, :, i:i + 1] = base + am
        sim = jnp.where(cand == am, _NEG, sim)


def _merge_body(ncand, M, vals_ref, idx_ref, oidx_ref, flag_ref=None):
    v = vals_ref[...]
    ids = idx_ref[...]
    iot = lax.broadcasted_iota(jnp.int32, v.shape, 1)
    if flag_ref is not None:
        tmax = jnp.max(jnp.where((iot & (M - 1)) == M - 1, v, _NEG), axis=1,
                       keepdims=True)
    mval = None
    for i in range(_TOPK):
        mval = jnp.max(v, axis=1, keepdims=True)
        cand = jnp.where(v == mval, iot, ncand)
        am = jnp.min(cand, axis=1, keepdims=True)
        gi = jnp.max(jnp.where(iot == am, ids, -1), axis=1, keepdims=True)
        oidx_ref[:, i:i + 1] = gi
        v = jnp.where(iot == am, _NEG, v)
    if flag_ref is not None:
        flag_ref[...] = jnp.broadcast_to(
            (tmax >= mval).astype(jnp.int32), flag_ref.shape)


def _sc_gather(table, idx_flat):
    info = plsc.get_sparse_core_info()
    nw = info.num_cores * info.num_subcores
    b_total = idx_flat.shape[0]
    d = table.shape[1]
    bpw = b_total // nw
    chunk = 80
    nchunk = bpw // chunk
    mesh = plsc.VectorSubcoreMesh(core_axis_name="c", subcore_axis_name="s")

    @functools.partial(
        pl.kernel, mesh=mesh,
        out_type=jax.ShapeDtypeStruct((b_total, d), jnp.float32),
        scratch_types=[
            pltpu.VMEM((bpw,), jnp.int32),
            pltpu.VMEM((bpw, d), jnp.float32),
            pltpu.SemaphoreType.DMA,
        ],
        compiler_params=pltpu.CompilerParams(use_tc_tiling_on_sc=False),
    )
    def gather_kernel(table_hbm, idx_hbm, out_hbm, idx_v, rows_v, sem):
        wid = lax.axis_index("s") * info.num_cores + lax.axis_index("c")
        base = wid * bpw
        pltpu.sync_copy(idx_hbm.at[pl.ds(base, bpw)], idx_v)
        copies = []
        for c in range(nchunk):
            copies.append(pltpu.async_copy(
                table_hbm.at[idx_v.at[pl.ds(c * chunk, chunk)]],
                rows_v.at[pl.ds(c * chunk, chunk)], sem))
        for cp in copies:
            cp.wait()
        pltpu.sync_copy(rows_v, out_hbm.at[pl.ds(base, bpw)])

    return gather_kernel(table, idx_flat)


def _post_body(g_ref, q_ref, wqt_ref, bq_ref, wmt_ref, bm_ref, lns_ref,
               lnb_ref, wcq_ref, wco_ref, bc_ref, out_ref):
    q = jnp.maximum(q_ref[...], 0.0)
    qt = jnp.maximum(
        jnp.dot(q, wqt_ref[...], preferred_element_type=jnp.float32,
                precision=lax.Precision.HIGHEST) + bq_ref[...], 0.0)
    x = g_ref[...].reshape(_TOPK * _QT3, q.shape[1])
    mt = jnp.maximum(
        jnp.dot(x, wmt_ref[...], preferred_element_type=jnp.float32,
                precision=lax.Precision.HIGHEST) + bm_ref[...], 0.0)
    acc = qt
    for j in range(_TOPK):
        acc = acc + mt[j * _QT3:(j + 1) * _QT3, :]
    mu = jnp.mean(acc, axis=1, keepdims=True)
    var = jnp.mean((acc - mu) ** 2, axis=1, keepdims=True)
    out = (acc - mu) / jnp.sqrt(var + 1e-5) * lns_ref[...] + lnb_ref[...]
    logits = (jnp.dot(q, wcq_ref[...], preferred_element_type=jnp.float32,
                      precision=lax.Precision.HIGHEST)
              + jnp.dot(out, wco_ref[...], preferred_element_type=jnp.float32,
                        precision=lax.Precision.HIGHEST)
              + bc_ref[...])
    out_ref[...] = logits


def kernel(query, memory_keys, Wq, bq, Wm, bm, ln_scale, ln_bias, Wc, bc, k):
    Q, D = query.shape
    K = memory_keys.shape[0]
    U = Wq.shape[0]
    C = Wc.shape[0]
    nkb = -(-K // _KB)

    def run_stage1(M):
        return pl.pallas_call(
            functools.partial(_stage1_body, K, M, nkb),
            grid=(Q // _QT, nkb),
            in_specs=[
                pl.BlockSpec((_QT, D), lambda i, j: (i, 0)),
                pl.BlockSpec((_KB, D), lambda i, j: (j, 0)),
            ],
            out_specs=[
                pl.BlockSpec((1, _QT, M), lambda i, j: (j, i, 0)),
                pl.BlockSpec((1, _QT, M), lambda i, j: (j, i, 0)),
            ],
            out_shape=[
                jax.ShapeDtypeStruct((nkb, Q, M), jnp.float32),
                jax.ShapeDtypeStruct((nkb, Q, M), jnp.int32),
            ],
            compiler_params=pltpu.CompilerParams(
                dimension_semantics=("parallel", "parallel")),
        )(query, memory_keys)

    def run_merge(vals, idxs, M, certify):
        ncand = nkb * M
        vals = jnp.swapaxes(vals, 0, 1).reshape(Q, ncand)
        idxs = jnp.swapaxes(idxs, 0, 1).reshape(Q, ncand)
        out_shape = [jax.ShapeDtypeStruct((Q, _TOPK), jnp.int32)]
        out_specs = [pl.BlockSpec((_QT2, _TOPK), lambda i: (i, 0))]
        if certify:
            out_shape.append(jax.ShapeDtypeStruct((Q, 128), jnp.int32))
            out_specs.append(pl.BlockSpec((_QT2, 128), lambda i: (i, 0)))
        return pl.pallas_call(
            functools.partial(_merge_body, ncand, M),
            grid=(Q // _QT2,),
            in_specs=[
                pl.BlockSpec((_QT2, ncand), lambda i: (i, 0)),
                pl.BlockSpec((_QT2, ncand), lambda i: (i, 0)),
            ],
            out_specs=out_specs,
            out_shape=out_shape,
        )(vals, idxs)

    vals_f, idxs_f = run_stage1(_MFAST)
    topi_fast, flags = run_merge(vals_f, idxs_f, _MFAST, certify=True)

    def full_path():
        vals50, idxs50 = run_stage1(_TOPK)
        (topi50,) = run_merge(vals50, idxs50, _TOPK, certify=False)
        return topi50

    topi = lax.cond(jnp.max(flags) > 0, full_path, lambda: topi_fast)

    idx_flat = jnp.transpose(topi).reshape(-1)
    g = _sc_gather(memory_keys, idx_flat)
    g3 = g.reshape(_TOPK, Q, D)

    logits = pl.pallas_call(
        _post_body,
        grid=(Q // _QT3,),
        in_specs=[
            pl.BlockSpec((_TOPK, _QT3, D), lambda i: (0, i, 0)),
            pl.BlockSpec((_QT3, D), lambda i: (i, 0)),
            pl.BlockSpec((D, U), lambda i: (0, 0)),
            pl.BlockSpec((1, U), lambda i: (0, 0)),
            pl.BlockSpec((D, U), lambda i: (0, 0)),
            pl.BlockSpec((1, U), lambda i: (0, 0)),
            pl.BlockSpec((1, U), lambda i: (0, 0)),
            pl.BlockSpec((1, U), lambda i: (0, 0)),
            pl.BlockSpec((D, C), lambda i: (0, 0)),
            pl.BlockSpec((U, C), lambda i: (0, 0)),
            pl.BlockSpec((1, C), lambda i: (0, 0)),
        ],
        out_specs=pl.BlockSpec((_QT3, C), lambda i: (i, 0)),
        out_shape=jax.ShapeDtypeStruct((Q, C), jnp.float32),
    )(g3, query, jnp.transpose(Wq), bq[None, :], jnp.transpose(Wm),
      bm[None, :], ln_scale[None, :], ln_bias[None, :],
      jnp.transpose(Wc[:, :D]), jnp.transpose(Wc[:, D:]), bc[None, :])
    return logits
